# baseline (speedup 1.0000x reference)
import functools

import jax
import jax.numpy as jnp
from jax import lax
from jax.experimental import pallas as pl
from jax.experimental.pallas import tpu as pltpu
from jax.experimental.pallas import tpu_sc as plsc

N_NODES = 10000
N_EDGES = 160000
NFEAT = 256
HID = 16
NCLASS = 40

FW = 48
NC = 2
NS = 16
NW = NC * NS
CHUNK = 128
CPP = 80
K0 = 56
K1 = CPP - K0
MAXK = max(K0, K1)
NROWS = NS * CPP
E_PAD = NROWS * CHUNK
STAGE_PAD = MAXK
RPT = N_NODES // NS

_mesh = plsc.VectorSubcoreMesh(
    core_axis_name="c", subcore_axis_name="s", num_cores=NC, num_subcores=NS
)


def _sc_body(table, srcr, dstr, wr, zeros, out, src_v, dst_v, w_v, rows_a,
             rows_b, rows_c, rows_d, sc_a, sc_b, sc_c, sc_d, acc, gsa, gsb,
             gsc, gsd, ssa, ssb, ssc, ssd):
    cid = lax.axis_index("c")
    sid = lax.axis_index("s")

    pltpu.sync_copy(zeros.at[pl.ds(sid * RPT, RPT)], acc.at[pl.ds(sid * RPT, RPT)])

    base = sid * CPP + jnp.where(cid == 0, 0, K0)
    nquad = jnp.where(cid == 0, K0 // 4, K1 // 4)
    pltpu.sync_copy(srcr.at[pl.ds(base, MAXK)], src_v)
    pltpu.sync_copy(dstr.at[pl.ds(base, MAXK)], dst_v)
    pltpu.sync_copy(wr.at[pl.ds(base * CHUNK, MAXK * CHUNK)], w_v)
    plsc.subcore_barrier()

    def _scale(rows, outr, j):
        jbase = j * CHUNK
        for g in range(CHUNK // 16):
            wv = w_v[pl.ds(jbase + g * 16, 16)]
            for l in range(16):
                e = g * 16 + l
                wb = lax.gather(
                    wv, jnp.full((16, 1), l, jnp.int32),
                    dimension_numbers=lax.GatherDimensionNumbers(
                        offset_dims=(), collapsed_slice_dims=(0,),
                        start_index_map=(0,)),
                    slice_sizes=(1,),
                    mode=lax.GatherScatterMode.PROMISE_IN_BOUNDS)
                for k in range(FW // 16):
                    sl = pl.ds(k * 16, 16)
                    outr[e, sl] = rows[e, sl] * wb

    bufs = (rows_a, rows_b, rows_c, rows_d)
    sbufs = (sc_a, sc_b, sc_c, sc_d)
    gsems = (gsa, gsb, gsc, gsd)
    ssems = (ssa, ssb, ssc, ssd)
    for b in range(4):
        pltpu.async_copy(table.at[src_v.at[b]], bufs[b], gsems[b])

    def quad(t, carry):
        c0 = 4 * t
        for b in range(4):
            c = c0 + b
            pltpu.make_async_copy(table.at[src_v.at[c]], bufs[b], gsems[b]).wait()
            @pl.when(t > 0)
            def _drain():
                pltpu.make_async_copy(
                    sbufs[b], acc.at[dst_v.at[c]], ssems[b]).wait()

            _scale(bufs[b], sbufs[b], c)

            @pl.when(t < nquad - 1)
            def _refill():
                pltpu.async_copy(table.at[src_v.at[c + 4]], bufs[b], gsems[b])

            pltpu.async_copy(sbufs[b], acc.at[dst_v.at[c]], ssems[b],
                             add=True)

        return carry

    lax.fori_loop(0, nquad, quad, 0)
    for b in range(4):
        pltpu.make_async_copy(sbufs[b], acc.at[dst_v.at[0]], ssems[b]).wait()

    plsc.subcore_barrier()
    pltpu.sync_copy(
        acc.at[pl.ds(sid * RPT, RPT)], out.at[cid, pl.ds(sid * RPT, RPT)]
    )


_sc_pass = functools.partial(
    pl.kernel,
    out_type=jax.ShapeDtypeStruct((NC, N_NODES, FW), jnp.float32),
    mesh=_mesh,
    scratch_types=[
        pltpu.VMEM((MAXK, CHUNK), jnp.int32),
        pltpu.VMEM((MAXK, CHUNK), jnp.int32),
        pltpu.VMEM((MAXK * CHUNK,), jnp.float32),
        pltpu.VMEM((CHUNK, FW), jnp.float32),
        pltpu.VMEM((CHUNK, FW), jnp.float32),
        pltpu.VMEM((CHUNK, FW), jnp.float32),
        pltpu.VMEM((CHUNK, FW), jnp.float32),
        pltpu.VMEM((CHUNK, FW), jnp.float32),
        pltpu.VMEM((CHUNK, FW), jnp.float32),
        pltpu.VMEM((CHUNK, FW), jnp.float32),
        pltpu.VMEM((CHUNK, FW), jnp.float32),
        pltpu.VMEM_SHARED((N_NODES, FW), jnp.float32),
        pltpu.SemaphoreType.DMA,
        pltpu.SemaphoreType.DMA,
        pltpu.SemaphoreType.DMA,
        pltpu.SemaphoreType.DMA,
        pltpu.SemaphoreType.DMA,
        pltpu.SemaphoreType.DMA,
        pltpu.SemaphoreType.DMA,
        pltpu.SemaphoreType.DMA,
    ],
    compiler_params=pltpu.CompilerParams(
        use_tc_tiling_on_sc=False, needs_layout_passes=False
    ),
)(_sc_body)


def _stage_a_body(x_ref, w_ref, b_ref, o_ref):
    o_ref[...] = (
        jnp.dot(x_ref[...], w_ref[...],
                preferred_element_type=jnp.float32,
                precision=lax.Precision.HIGHEST)
        + b_ref[...]
    )


def _stage_c_body(p_ref, b1_ref, bns_ref, w3_ref, o_ref):
    agg = p_ref[0] + p_ref[1]
    x1 = jnp.maximum(agg[:, 0:HID] + b1_ref[...], 0.0)
    deg = agg[:, 2 * HID:2 * HID + 1]
    x2 = jnp.maximum(agg[:, HID:2 * HID] / (deg + 1e-6) + bns_ref[...], 0.0)
    h = jnp.concatenate([x1, x2], axis=1)
    o_ref[...] = jnp.dot(h, w3_ref[...],
                         preferred_element_type=jnp.float32,
                         precision=lax.Precision.HIGHEST)


def _stage_e_body(p_ref, b3_ref, o_ref):
    agg = p_ref[0] + p_ref[1]
    logits = agg[:, 0:NCLASS] + b3_ref[...]
    m = jnp.max(logits, axis=1, keepdims=True)
    s = logits - m
    o_ref[...] = s - jnp.log(jnp.sum(jnp.exp(s), axis=1, keepdims=True))


_BLK = 2000


def _stage_a(x, wc, brow):
    return pl.pallas_call(
        _stage_a_body,
        grid=(N_NODES // _BLK,),
        in_specs=[
            pl.BlockSpec((_BLK, NFEAT), lambda i: (i, 0)),
            pl.BlockSpec((NFEAT, FW), lambda i: (0, 0)),
            pl.BlockSpec((1, FW), lambda i: (0, 0)),
        ],
        out_specs=pl.BlockSpec((_BLK, FW), lambda i: (i, 0)),
        out_shape=jax.ShapeDtypeStruct((N_NODES, FW), jnp.float32),
    )(x, wc, brow)


def _stage_c(p, b1r, bnsr, w3p):
    return pl.pallas_call(
        _stage_c_body,
        grid=(N_NODES // _BLK,),
        in_specs=[
            pl.BlockSpec((NC, _BLK, FW), lambda i: (0, i, 0)),
            pl.BlockSpec((1, HID), lambda i: (0, 0)),
            pl.BlockSpec((1, HID), lambda i: (0, 0)),
            pl.BlockSpec((2 * HID, FW), lambda i: (0, 0)),
        ],
        out_specs=pl.BlockSpec((_BLK, FW), lambda i: (i, 0)),
        out_shape=jax.ShapeDtypeStruct((N_NODES, FW), jnp.float32),
    )(p, b1r, bnsr, w3p)


def _stage_e(p, b3r):
    return pl.pallas_call(
        _stage_e_body,
        grid=(N_NODES // _BLK,),
        in_specs=[
            pl.BlockSpec((NC, _BLK, FW), lambda i: (0, i, 0)),
            pl.BlockSpec((1, NCLASS), lambda i: (0, 0)),
        ],
        out_specs=pl.BlockSpec((_BLK, NCLASS), lambda i: (i, 0)),
        out_shape=jax.ShapeDtypeStruct((N_NODES, NCLASS), jnp.float32),
    )(p, b3r)


def kernel(x, adj, adj_weight, W1, b1, Wns, bns, W3, b3):
    src = adj[0].astype(jnp.int32)
    dst = adj[1].astype(jnp.int32)
    pad = (NROWS + STAGE_PAD) * CHUNK - N_EDGES
    srcr = jnp.concatenate([src, jnp.zeros((pad,), jnp.int32)]).reshape(
        NROWS + STAGE_PAD, CHUNK)
    dstr = jnp.concatenate([dst, jnp.zeros((pad,), jnp.int32)]).reshape(
        NROWS + STAGE_PAD, CHUNK)
    wr = jnp.concatenate([adj_weight, jnp.zeros((pad,), jnp.float32)])

    wc = jnp.zeros((NFEAT, FW), jnp.float32)
    wc = wc.at[:, 0:HID].set(W1).at[:, HID:2 * HID].set(Wns)
    brow = jnp.zeros((1, FW), jnp.float32).at[0, 2 * HID].set(1.0)
    zeros = jnp.zeros((N_NODES, FW), jnp.float32)

    table1 = _stage_a(x, wc, brow)
    p1 = _sc_pass(table1, srcr, dstr, wr, zeros)
    w3p = jnp.zeros((2 * HID, FW), jnp.float32).at[:, 0:NCLASS].set(W3)
    table2 = _stage_c(p1, b1.reshape(1, HID), bns.reshape(1, HID), w3p)
    p2 = _sc_pass(table2, srcr, dstr, wr, zeros)
    return _stage_e(p2, b3.reshape(1, NCLASS))


# speedup vs baseline: 1.0096x; 1.0096x over previous
import functools

import jax
import jax.numpy as jnp
from jax import lax
from jax.experimental import pallas as pl
from jax.experimental.pallas import tpu as pltpu
from jax.experimental.pallas import tpu_sc as plsc

N_NODES = 10000
N_EDGES = 160000
NFEAT = 256
HID = 16
NCLASS = 40

FW = 48
NC = 2
NS = 16
NW = NC * NS
CHUNK = 128
CPP = 80
K0 = 56
K1 = CPP - K0
MAXK = max(K0, K1)
NROWS = NS * CPP
E_PAD = NROWS * CHUNK
STAGE_PAD = MAXK
RPT = N_NODES // NS

_mesh = plsc.VectorSubcoreMesh(
    core_axis_name="c", subcore_axis_name="s", num_cores=NC, num_subcores=NS
)


def _sc_body(table, srcr, dstr, wr, zeros, out, src_v, dst_v, w_v, rows_a,
             rows_b, rows_c, rows_d, acc, gsa, gsb, gsc, gsd, ssa, ssb,
             ssc, ssd):
    cid = lax.axis_index("c")
    sid = lax.axis_index("s")

    pltpu.sync_copy(zeros.at[pl.ds(sid * RPT, RPT)], acc.at[pl.ds(sid * RPT, RPT)])

    base = sid * CPP + jnp.where(cid == 0, 0, K0)
    nquad = jnp.where(cid == 0, K0 // 4, K1 // 4)
    pltpu.sync_copy(srcr.at[pl.ds(base, MAXK)], src_v)
    pltpu.sync_copy(dstr.at[pl.ds(base, MAXK)], dst_v)
    pltpu.sync_copy(wr.at[pl.ds(base * CHUNK, MAXK * CHUNK)], w_v)
    plsc.subcore_barrier()

    def _scale(rows, j):
        jbase = j * CHUNK
        for g in range(CHUNK // 16):
            wv = w_v[pl.ds(jbase + g * 16, 16)]
            for l in range(16):
                e = g * 16 + l
                wb = lax.gather(
                    wv, jnp.full((16, 1), l, jnp.int32),
                    dimension_numbers=lax.GatherDimensionNumbers(
                        offset_dims=(), collapsed_slice_dims=(0,),
                        start_index_map=(0,)),
                    slice_sizes=(1,),
                    mode=lax.GatherScatterMode.PROMISE_IN_BOUNDS)
                for k in range(FW // 16):
                    sl = pl.ds(k * 16, 16)
                    rows[e, sl] = rows[e, sl] * wb

    bufs = (rows_a, rows_b, rows_c, rows_d)
    gsems = (gsa, gsb, gsc, gsd)
    ssems = (ssa, ssb, ssc, ssd)
    for b in range(4):
        pltpu.async_copy(table.at[src_v.at[b]], bufs[b], gsems[b])

    def quad(t, carry):
        c0 = 4 * t
        for b in range(4):
            c = c0 + b
            pltpu.make_async_copy(table.at[src_v.at[c]], bufs[b], gsems[b]).wait()
            _scale(bufs[b], c)
            sc = pltpu.async_copy(bufs[b], acc.at[dst_v.at[c]], ssems[b],
                                  add=True)
            sc.wait()

            @pl.when(t < nquad - 1)
            def _refill():
                pltpu.async_copy(table.at[src_v.at[c + 4]], bufs[b], gsems[b])

        return carry

    lax.fori_loop(0, nquad, quad, 0)

    plsc.subcore_barrier()
    pltpu.sync_copy(
        acc.at[pl.ds(sid * RPT, RPT)], out.at[cid, pl.ds(sid * RPT, RPT)]
    )


_sc_pass = functools.partial(
    pl.kernel,
    out_type=jax.ShapeDtypeStruct((NC, N_NODES, FW), jnp.float32),
    mesh=_mesh,
    scratch_types=[
        pltpu.VMEM((MAXK, CHUNK), jnp.int32),
        pltpu.VMEM((MAXK, CHUNK), jnp.int32),
        pltpu.VMEM((MAXK * CHUNK,), jnp.float32),
        pltpu.VMEM((CHUNK, FW), jnp.float32),
        pltpu.VMEM((CHUNK, FW), jnp.float32),
        pltpu.VMEM((CHUNK, FW), jnp.float32),
        pltpu.VMEM((CHUNK, FW), jnp.float32),
        pltpu.VMEM_SHARED((N_NODES, FW), jnp.float32),
        pltpu.SemaphoreType.DMA,
        pltpu.SemaphoreType.DMA,
        pltpu.SemaphoreType.DMA,
        pltpu.SemaphoreType.DMA,
        pltpu.SemaphoreType.DMA,
        pltpu.SemaphoreType.DMA,
        pltpu.SemaphoreType.DMA,
        pltpu.SemaphoreType.DMA,
    ],
    compiler_params=pltpu.CompilerParams(
        use_tc_tiling_on_sc=False, needs_layout_passes=False
    ),
)(_sc_body)


def _stage_a_body(x_ref, w_ref, b_ref, o_ref):
    o_ref[...] = (
        jnp.dot(x_ref[...], w_ref[...],
                preferred_element_type=jnp.float32,
                precision=lax.Precision.HIGHEST)
        + b_ref[...]
    )


def _stage_c_body(p_ref, b1_ref, bns_ref, w3_ref, o_ref):
    agg = p_ref[0] + p_ref[1]
    x1 = jnp.maximum(agg[:, 0:HID] + b1_ref[...], 0.0)
    deg = agg[:, 2 * HID:2 * HID + 1]
    x2 = jnp.maximum(agg[:, HID:2 * HID] / (deg + 1e-6) + bns_ref[...], 0.0)
    h = jnp.concatenate([x1, x2], axis=1)
    o_ref[...] = jnp.dot(h, w3_ref[...],
                         preferred_element_type=jnp.float32,
                         precision=lax.Precision.HIGHEST)


def _stage_e_body(p_ref, b3_ref, o_ref):
    agg = p_ref[0] + p_ref[1]
    logits = agg[:, 0:NCLASS] + b3_ref[...]
    m = jnp.max(logits, axis=1, keepdims=True)
    s = logits - m
    o_ref[...] = s - jnp.log(jnp.sum(jnp.exp(s), axis=1, keepdims=True))


_BLK = 2000


def _stage_a(x, wc, brow):
    return pl.pallas_call(
        _stage_a_body,
        grid=(N_NODES // _BLK,),
        in_specs=[
            pl.BlockSpec((_BLK, NFEAT), lambda i: (i, 0)),
            pl.BlockSpec((NFEAT, FW), lambda i: (0, 0)),
            pl.BlockSpec((1, FW), lambda i: (0, 0)),
        ],
        out_specs=pl.BlockSpec((_BLK, FW), lambda i: (i, 0)),
        out_shape=jax.ShapeDtypeStruct((N_NODES, FW), jnp.float32),
    )(x, wc, brow)


def _stage_c(p, b1r, bnsr, w3p):
    return pl.pallas_call(
        _stage_c_body,
        grid=(N_NODES // _BLK,),
        in_specs=[
            pl.BlockSpec((NC, _BLK, FW), lambda i: (0, i, 0)),
            pl.BlockSpec((1, HID), lambda i: (0, 0)),
            pl.BlockSpec((1, HID), lambda i: (0, 0)),
            pl.BlockSpec((2 * HID, FW), lambda i: (0, 0)),
        ],
        out_specs=pl.BlockSpec((_BLK, FW), lambda i: (i, 0)),
        out_shape=jax.ShapeDtypeStruct((N_NODES, FW), jnp.float32),
    )(p, b1r, bnsr, w3p)


def _stage_e(p, b3r):
    return pl.pallas_call(
        _stage_e_body,
        grid=(N_NODES // _BLK,),
        in_specs=[
            pl.BlockSpec((NC, _BLK, FW), lambda i: (0, i, 0)),
            pl.BlockSpec((1, NCLASS), lambda i: (0, 0)),
        ],
        out_specs=pl.BlockSpec((_BLK, NCLASS), lambda i: (i, 0)),
        out_shape=jax.ShapeDtypeStruct((N_NODES, NCLASS), jnp.float32),
    )(p, b3r)


def kernel(x, adj, adj_weight, W1, b1, Wns, bns, W3, b3):
    src = adj[0].astype(jnp.int32)
    dst = adj[1].astype(jnp.int32)
    pad = (NROWS + STAGE_PAD) * CHUNK - N_EDGES
    srcr = jnp.concatenate([src, jnp.zeros((pad,), jnp.int32)]).reshape(
        NROWS + STAGE_PAD, CHUNK)
    dstr = jnp.concatenate([dst, jnp.zeros((pad,), jnp.int32)]).reshape(
        NROWS + STAGE_PAD, CHUNK)
    wr = jnp.concatenate([adj_weight, jnp.zeros((pad,), jnp.float32)])

    wc = jnp.zeros((NFEAT, FW), jnp.float32)
    wc = wc.at[:, 0:HID].set(W1).at[:, HID:2 * HID].set(Wns)
    brow = jnp.zeros((1, FW), jnp.float32).at[0, 2 * HID].set(1.0)
    zeros = jnp.zeros((N_NODES, FW), jnp.float32)

    table1 = _stage_a(x, wc, brow)
    p1 = _sc_pass(table1, srcr, dstr, wr, zeros)
    w3p = jnp.zeros((2 * HID, FW), jnp.float32).at[:, 0:NCLASS].set(W3)
    table2 = _stage_c(p1, b1.reshape(1, HID), bns.reshape(1, HID), w3p)
    p2 = _sc_pass(table2, srcr, dstr, wr, zeros)
    return _stage_e(p2, b3.reshape(1, NCLASS))
